# baseline (speedup 1.0000x reference)
import functools

import jax
import jax.numpy as jnp
from jax import lax
from jax.experimental import pallas as pl
from jax.experimental.pallas import tpu as pltpu
from jax.experimental.pallas import tpu_sc as plsc

N = 10000
NP = 10240
E = 160000
EP = 172032
C = 4096
NB = 1024
F32 = jnp.float32

_mesh = plsc.VectorSubcoreMesh(core_axis_name="c", subcore_axis_name="s")
_CP_SC = pltpu.CompilerParams(needs_layout_passes=False)


def _tc_pre_body(x_ref, w_ref, a_ref, hT_ref, al_ref, gmax_ref):
    hT = lax.dot_general(w_ref[...], x_ref[...], (((0,), (1,)), ((), ())),
                         preferred_element_type=F32)
    hT_ref[...] = hT
    al = lax.dot_general(a_ref[...], hT, (((0,), (0,)), ((), ())),
                         preferred_element_type=F32)
    al_ref[...] = al
    rm = jnp.max(al, axis=1, keepdims=True)
    rmb = lax.broadcast_in_dim(rm, (8, 128), (0, 1))

    @pl.when(pl.program_id(0) == 0)
    def _():
        gmax_ref[...] = rmb

    @pl.when(pl.program_id(0) != 0)
    def _():
        gmax_ref[...] = jnp.maximum(gmax_ref[...], rmb)


def _tc_pre(xp, W, A, dh):
    din = xp.shape[1]
    return pl.pallas_call(
        _tc_pre_body,
        grid=(NP // NB,),
        in_specs=[
            pl.BlockSpec((NB, din), lambda i: (i, 0)),
            pl.BlockSpec((din, dh), lambda i: (0, 0)),
            pl.BlockSpec((dh, 8), lambda i: (0, 0)),
        ],
        out_specs=[
            pl.BlockSpec((dh, NB), lambda i: (0, i)),
            pl.BlockSpec((8, NB), lambda i: (0, i)),
            pl.BlockSpec((8, 128), lambda i: (0, 0)),
        ],
        out_shape=[
            jax.ShapeDtypeStruct((dh, NP), F32),
            jax.ShapeDtypeStruct((8, NP), F32),
            jax.ShapeDtypeStruct((8, 128), F32),
        ],
    )(xp, W, A)


def _tc_mid_body(acc_ref, den_ref, b_ref, w_ref, a_ref,
                 zT_ref, al_ref, gmax_ref):
    i = pl.program_id(0)
    acc = acc_ref[...]
    den = den_ref[0:4, :]
    col = lax.broadcasted_iota(jnp.int32, (1, NB), 1) + i * NB
    valid = col < N
    acc = jnp.where(lax.broadcast_in_dim(valid, (256, NB), (0, 1)), acc, 0.0)
    den = jnp.where(lax.broadcast_in_dim(valid, (4, NB), (0, 1)), den, 1.0)
    acc3 = acc.reshape(4, 64, NB)
    den3 = lax.broadcast_in_dim(den, (4, 64, NB), (0, 2))
    h = acc3 / (den3 + 1e-16) + b_ref[...].reshape(4, 64, 1)
    h = h.reshape(256, NB)
    h = jnp.where(h > 0, h, jnp.exp(h) - 1.0)
    z = lax.dot_general(w_ref[...], h, (((0,), (0,)), ((), ())),
                        preferred_element_type=F32)
    zT_ref[...] = z
    al2 = lax.dot_general(a_ref[...], z, (((0,), (0,)), ((), ())),
                          preferred_element_type=F32)
    al2p = jnp.concatenate([al2, jnp.full((6, NB), -1e30, F32)], axis=0)
    al_ref[...] = al2p
    rm = jnp.max(al2p, axis=1, keepdims=True)
    rmb = lax.broadcast_in_dim(rm, (8, 128), (0, 1))

    @pl.when(i == 0)
    def _():
        gmax_ref[...] = rmb

    @pl.when(i != 0)
    def _():
        gmax_ref[...] = jnp.maximum(gmax_ref[...], rmb)


def _tc_mid(accT, denT, b1c, W2, A2):
    return pl.pallas_call(
        _tc_mid_body,
        grid=(NP // NB,),
        in_specs=[
            pl.BlockSpec((256, NB), lambda i: (0, i)),
            pl.BlockSpec((8, NB), lambda i: (0, i)),
            pl.BlockSpec((256, 1), lambda i: (0, 0)),
            pl.BlockSpec((256, 128), lambda i: (0, 0)),
            pl.BlockSpec((128, 2), lambda i: (0, 0)),
        ],
        out_specs=[
            pl.BlockSpec((128, NB), lambda i: (0, i)),
            pl.BlockSpec((8, NB), lambda i: (0, i)),
            pl.BlockSpec((8, 128), lambda i: (0, 0)),
        ],
        out_shape=[
            jax.ShapeDtypeStruct((128, NP), F32),
            jax.ShapeDtypeStruct((8, NP), F32),
            jax.ShapeDtypeStruct((8, 128), F32),
        ],
    )(accT, denT, b1c, W2, A2)


def _tc_post_body(acc_ref, den_ref, b_ref, eye_ref, out_ref):
    acc = acc_ref[...]
    den = den_ref[0:1, :]
    h = acc / (lax.broadcast_in_dim(den, (128, NB), (0, 1)) + 1e-16)
    h = h + b_ref[...]
    h = jnp.where(h > 0, h, jnp.exp(h) - 1.0)
    out_ref[...] = lax.dot_general(h, eye_ref[...], (((0,), (0,)), ((), ())),
                                   preferred_element_type=F32)


def _tc_post(acc2T, den2, b2c, eye):
    return pl.pallas_call(
        _tc_post_body,
        grid=(NP // NB,),
        in_specs=[
            pl.BlockSpec((128, NB), lambda i: (0, i)),
            pl.BlockSpec((8, NB), lambda i: (0, i)),
            pl.BlockSpec((128, 1), lambda i: (0, 0)),
            pl.BlockSpec((128, 128), lambda i: (0, 0)),
        ],
        out_specs=pl.BlockSpec((NB, 128), lambda i: (i, 0)),
        out_shape=jax.ShapeDtypeStruct((NP, 128), F32),
    )(acc2T, den2, b2c, eye)


def _make_edge_kernel(heads, featc):
    nunits = featc // 4
    units_per_tec = nunits // 32
    dst_row = 4 if heads == 4 else 1
    chunks = EP // C

    @functools.partial(
        pl.kernel,
        out_type=(jax.ShapeDtypeStruct((featc * NP,), F32),
                  jax.ShapeDtypeStruct((8 * NP,), F32),
                  jax.ShapeDtypeStruct((featc * NP,), F32),
                  jax.ShapeDtypeStruct((8 * NP,), F32)),
        mesh=_mesh,
        compiler_params=_CP_SC,
        scratch_types=(
            [pltpu.VMEM((NP,), F32) for _ in range(4)]
            + [pltpu.VMEM((NP,), F32) for _ in range(4)]
            + [
                pltpu.VMEM((NP,), F32),
                pltpu.VMEM((NP,), F32),
                pltpu.VMEM((NP,), F32),
                pltpu.VMEM((C,), jnp.int32),
                pltpu.VMEM((C,), jnp.int32),
                pltpu.VMEM((C,), jnp.int32),
                pltpu.VMEM((C,), jnp.int32),
                pltpu.VMEM((128,), F32),
                pltpu.VMEM((128,), F32),
                pltpu.SemaphoreType.DMA,
                pltpu.SemaphoreType.DMA,
                pltpu.SemaphoreType.DMA,
                pltpu.SemaphoreType.DMA,
            ]
        ),
    )
    def edge_kernel(tabT_a, alphaT_a, gmaxrow_a, src_a, dst_a,
                    tabT_b, alphaT_b, gmaxrow_b, src_b, dst_b,
                    accT_oa, den_oa, accT_ob, den_ob,
                    t0, t1, t2, t3, a0, a1, a2, a3,
                    asr, ads, accd, sv0, dv0, sv1, dv1, gm1, gm2,
                    ss0, sd0, ss1, sd1):
        tabs = (t0, t1, t2, t3)
        accs = (a0, a1, a2, a3)
        cid = lax.axis_index("c")
        sid = lax.axis_index("s")
        wid = sid * 2 + cid
        zeros = jnp.zeros((16,), F32)
        graphs = (
            (tabT_a, alphaT_a, gmaxrow_a, src_a, dst_a, accT_oa, den_oa),
            (tabT_b, alphaT_b, gmaxrow_b, src_b, dst_b, accT_ob, den_ob),
        )
        work = [g + (t,) for g in graphs for t in range(units_per_tec)]
        for tabT, alphaT, gmaxrow, src, dst, accT_o, den_o, t in work:
            u = wid * units_per_tec + t
            head = (u // 16) if heads == 4 else (u * 0)
            setup = []
            for f in range(4):
                setup.append(pltpu.async_copy(
                    tabT.at[pl.ds((u * 4 + f) * NP, NP)], tabs[f], ss0))
            setup.append(pltpu.async_copy(
                alphaT.at[pl.ds(head * NP, NP)], asr, sd0))
            setup.append(pltpu.async_copy(
                alphaT.at[pl.ds((dst_row + head) * NP, NP)], ads, ss1))
            setup.append(pltpu.async_copy(
                gmaxrow.at[pl.ds(head * 128, 128)], gm1, sd1))
            setup.append(pltpu.async_copy(
                gmaxrow.at[pl.ds((dst_row + head) * 128, 128)], gm2, sd1))

            @plsc.parallel_loop(0, NP, 16, unroll=8)
            def _zero(o):
                for f in range(4):
                    accs[f][pl.ds(o, 16)] = zeros
                accd[pl.ds(o, 16)] = zeros

            for d in setup:
                d.wait()
            b = gm1[pl.ds(0, 16)] + gm2[pl.ds(0, 16)]
            g = jnp.maximum(b, 0.2 * b)

            def _start(ci, svb, dvb, sems):
                pltpu.async_copy(src.at[pl.ds(ci * C, C)], svb, sems[0])
                pltpu.async_copy(dst.at[pl.ds(ci * C, C)], dvb, sems[1])

            def _wait(svb, dvb, sems):
                pltpu.make_async_copy(src.at[pl.ds(0, C)], svb, sems[0]).wait()
                pltpu.make_async_copy(dst.at[pl.ds(0, C)], dvb, sems[1]).wait()

            def _run(svb, dvb, with_den):
                @plsc.parallel_loop(0, C, 16, unroll=4)
                def _body(o):
                    s = svb[pl.ds(o, 16)]
                    d = dvb[pl.ds(o, 16)]
                    e = plsc.load_gather(asr, [s]) + plsc.load_gather(ads, [d])
                    e = jnp.maximum(e, 0.2 * e)
                    ex = jnp.exp(e - g)
                    for f in range(4):
                        tv = plsc.load_gather(tabs[f], [s])
                        plsc.addupdate_scatter(accs[f], [d], tv * ex)
                    if with_den:
                        plsc.addupdate_scatter(accd, [d], ex)

            def _edge_sweep(with_den):
                _start(0, sv0, dv0, (ss0, sd0))

                def _pair(j, carry):
                    ci = 2 * j
                    _start(ci + 1, sv1, dv1, (ss1, sd1))
                    _wait(sv0, dv0, (ss0, sd0))
                    _run(sv0, dv0, with_den)
                    _start(jnp.minimum(ci + 2, chunks - 1), sv0, dv0,
                           (ss0, sd0))
                    _wait(sv1, dv1, (ss1, sd1))
                    _run(sv1, dv1, with_den)
                    return carry

                lax.fori_loop(0, chunks // 2, _pair, 0)
                _wait(sv0, dv0, (ss0, sd0))

            if heads == 4:
                is_aug = ((u == 2) | (u == 16) | (u == 34) | (u == 48))
            else:
                is_aug = (u == 0)

            @pl.when(is_aug)
            def _():
                _edge_sweep(True)

            @pl.when(jnp.logical_not(is_aug))
            def _():
                _edge_sweep(False)

            for f in range(4):
                pltpu.sync_copy(accs[f], accT_o.at[pl.ds((u * 4 + f) * NP, NP)])

            @pl.when(is_aug)
            def _():
                pltpu.sync_copy(accd, den_o.at[pl.ds(head * NP, NP)])

    return edge_kernel


_edge_l1 = _make_edge_kernel(4, 256)
_edge_l2 = _make_edge_kernel(1, 128)


def _edges(edge_index):
    loop = jnp.arange(N, dtype=edge_index.dtype)
    src = jnp.concatenate([edge_index[0], loop])
    dst = jnp.concatenate([edge_index[1], loop])
    pad = jnp.full((EP - E - N,), NP - 1, dtype=src.dtype)
    return (jnp.concatenate([src, pad]), jnp.concatenate([dst, pad]))


def _attn_mats(as1, ad1):
    eye4 = jnp.eye(4, dtype=F32)
    A1s = jnp.einsum("hc,hk->hck", as1, eye4).reshape(256, 4)
    A1d = jnp.einsum("hc,hk->hck", ad1, eye4).reshape(256, 4)
    return jnp.concatenate([A1s, A1d], axis=1)


def kernel(x0, x1, edge_index0, edge_index1, W1_0, as1_0, ad1_0, b1_0, W2_0, as2_0, ad2_0, b2_0, W1_1, as1_1, ad1_1, b1_1, W2_1, as2_1, ad2_1, b2_1):
    src0, dst0 = _edges(edge_index0)
    src1, dst1 = _edges(edge_index1)
    xp0 = jnp.pad(x0, ((0, NP - N), (0, 0)))
    xp1 = jnp.pad(x1, ((0, NP - N), (0, 0)))
    A1_0 = _attn_mats(as1_0, ad1_0)
    A1_1 = _attn_mats(as1_1, ad1_1)
    A2_0 = jnp.stack([as2_0[0], ad2_0[0]], axis=1)
    A2_1 = jnp.stack([as2_1[0], ad2_1[0]], axis=1)
    eye = jnp.eye(128, dtype=F32)

    h1T_0, alphaT_0, gmax_0 = _tc_pre(xp0, W1_0, A1_0, 256)
    h1T_1, alphaT_1, gmax_1 = _tc_pre(xp1, W1_1, A1_1, 256)
    accT0, denT0, accT1, denT1 = _edge_l1(
        h1T_0.reshape(-1), alphaT_0.reshape(-1), gmax_0.reshape(-1),
        src0, dst0,
        h1T_1.reshape(-1), alphaT_1.reshape(-1), gmax_1.reshape(-1),
        src1, dst1)
    zT0, alphaT2_0, gmax2_0 = _tc_mid(accT0.reshape(256, NP),
                                      denT0.reshape(8, NP),
                                      b1_0[:, None], W2_0, A2_0)
    zT1, alphaT2_1, gmax2_1 = _tc_mid(accT1.reshape(256, NP),
                                      denT1.reshape(8, NP),
                                      b1_1[:, None], W2_1, A2_1)
    acc2T0, den20, acc2T1, den21 = _edge_l2(
        zT0.reshape(-1), alphaT2_0.reshape(-1), gmax2_0.reshape(-1),
        src0, dst0,
        zT1.reshape(-1), alphaT2_1.reshape(-1), gmax2_1.reshape(-1),
        src1, dst1)
    out0 = _tc_post(acc2T0.reshape(128, NP), den20.reshape(8, NP),
                    b2_0[:, None], eye)
    out1 = _tc_post(acc2T1.reshape(128, NP), den21.reshape(8, NP),
                    b2_1[:, None], eye)
    return jnp.concatenate([out0[:N], out1[:N]], axis=0)


# speedup vs baseline: 1.0720x; 1.0720x over previous
import functools

import jax
import jax.numpy as jnp
from jax import lax
from jax.experimental import pallas as pl
from jax.experimental.pallas import tpu as pltpu
from jax.experimental.pallas import tpu_sc as plsc

N = 10000
NP = 10240
E = 160000
EP = 172032
C = 4096
NB = 1024
F32 = jnp.float32

_mesh = plsc.VectorSubcoreMesh(core_axis_name="c", subcore_axis_name="s")
_CP_SC = pltpu.CompilerParams(needs_layout_passes=False)


def _tc_pre_body(x_ref, w_ref, a_ref, hT_ref, al_ref, gmax_ref):
    hT = lax.dot_general(w_ref[...], x_ref[...], (((0,), (1,)), ((), ())),
                         preferred_element_type=F32)
    hT_ref[...] = hT
    al = lax.dot_general(a_ref[...], hT, (((0,), (0,)), ((), ())),
                         preferred_element_type=F32)
    al_ref[...] = al
    rm = jnp.max(al, axis=1, keepdims=True)
    rmb = lax.broadcast_in_dim(rm, (8, 128), (0, 1))

    @pl.when(pl.program_id(0) == 0)
    def _():
        gmax_ref[...] = rmb

    @pl.when(pl.program_id(0) != 0)
    def _():
        gmax_ref[...] = jnp.maximum(gmax_ref[...], rmb)


def _tc_pre(xp, W, A, dh):
    din = xp.shape[1]
    return pl.pallas_call(
        _tc_pre_body,
        grid=(NP // NB,),
        in_specs=[
            pl.BlockSpec((NB, din), lambda i: (i, 0)),
            pl.BlockSpec((din, dh), lambda i: (0, 0)),
            pl.BlockSpec((dh, 8), lambda i: (0, 0)),
        ],
        out_specs=[
            pl.BlockSpec((dh, NB), lambda i: (0, i)),
            pl.BlockSpec((8, NB), lambda i: (0, i)),
            pl.BlockSpec((8, 128), lambda i: (0, 0)),
        ],
        out_shape=[
            jax.ShapeDtypeStruct((dh, NP), F32),
            jax.ShapeDtypeStruct((8, NP), F32),
            jax.ShapeDtypeStruct((8, 128), F32),
        ],
    )(xp, W, A)


def _tc_mid_body(acc_ref, den_ref, b_ref, w_ref, a_ref,
                 zT_ref, al_ref, gmax_ref):
    i = pl.program_id(0)
    acc = acc_ref[...]
    den = den_ref[0:4, :]
    col = lax.broadcasted_iota(jnp.int32, (1, NB), 1) + i * NB
    valid = col < N
    acc = jnp.where(lax.broadcast_in_dim(valid, (256, NB), (0, 1)), acc, 0.0)
    den = jnp.where(lax.broadcast_in_dim(valid, (4, NB), (0, 1)), den, 1.0)
    acc3 = acc.reshape(4, 64, NB)
    den3 = lax.broadcast_in_dim(den, (4, 64, NB), (0, 2))
    h = acc3 / (den3 + 1e-16) + b_ref[...].reshape(4, 64, 1)
    h = h.reshape(256, NB)
    h = jnp.where(h > 0, h, jnp.exp(h) - 1.0)
    z = lax.dot_general(w_ref[...], h, (((0,), (0,)), ((), ())),
                        preferred_element_type=F32)
    zT_ref[...] = z
    al2 = lax.dot_general(a_ref[...], z, (((0,), (0,)), ((), ())),
                          preferred_element_type=F32)
    al2p = jnp.concatenate([al2, jnp.full((6, NB), -1e30, F32)], axis=0)
    al_ref[...] = al2p
    rm = jnp.max(al2p, axis=1, keepdims=True)
    rmb = lax.broadcast_in_dim(rm, (8, 128), (0, 1))

    @pl.when(i == 0)
    def _():
        gmax_ref[...] = rmb

    @pl.when(i != 0)
    def _():
        gmax_ref[...] = jnp.maximum(gmax_ref[...], rmb)


def _tc_mid(accT, denT, b1c, W2, A2):
    return pl.pallas_call(
        _tc_mid_body,
        grid=(NP // NB,),
        in_specs=[
            pl.BlockSpec((256, NB), lambda i: (0, i)),
            pl.BlockSpec((8, NB), lambda i: (0, i)),
            pl.BlockSpec((256, 1), lambda i: (0, 0)),
            pl.BlockSpec((256, 128), lambda i: (0, 0)),
            pl.BlockSpec((128, 2), lambda i: (0, 0)),
        ],
        out_specs=[
            pl.BlockSpec((128, NB), lambda i: (0, i)),
            pl.BlockSpec((8, NB), lambda i: (0, i)),
            pl.BlockSpec((8, 128), lambda i: (0, 0)),
        ],
        out_shape=[
            jax.ShapeDtypeStruct((128, NP), F32),
            jax.ShapeDtypeStruct((8, NP), F32),
            jax.ShapeDtypeStruct((8, 128), F32),
        ],
    )(accT, denT, b1c, W2, A2)


def _tc_post_body(acc_ref, den_ref, b_ref, eye_ref, out_ref):
    acc = acc_ref[...]
    den = den_ref[0:1, :]
    h = acc / (lax.broadcast_in_dim(den, (128, NB), (0, 1)) + 1e-16)
    h = h + b_ref[...]
    h = jnp.where(h > 0, h, jnp.exp(h) - 1.0)
    out_ref[...] = lax.dot_general(h, eye_ref[...], (((0,), (0,)), ((), ())),
                                   preferred_element_type=F32)


def _tc_post(acc2T, den2, b2c, eye):
    return pl.pallas_call(
        _tc_post_body,
        grid=(NP // NB,),
        in_specs=[
            pl.BlockSpec((128, NB), lambda i: (0, i)),
            pl.BlockSpec((8, NB), lambda i: (0, i)),
            pl.BlockSpec((128, 1), lambda i: (0, 0)),
            pl.BlockSpec((128, 128), lambda i: (0, 0)),
        ],
        out_specs=pl.BlockSpec((NB, 128), lambda i: (i, 0)),
        out_shape=jax.ShapeDtypeStruct((NP, 128), F32),
    )(acc2T, den2, b2c, eye)


def _make_edge_kernel(heads, featc):
    nunits = featc // 4
    units_per_tec = nunits // 32
    dst_row = 4 if heads == 4 else 1
    chunks = EP // C

    @functools.partial(
        pl.kernel,
        out_type=(jax.ShapeDtypeStruct((featc * NP,), F32),
                  jax.ShapeDtypeStruct((8 * NP,), F32)),
        mesh=_mesh,
        compiler_params=_CP_SC,
        scratch_types=(
            [pltpu.VMEM((NP,), F32) for _ in range(4)]
            + [pltpu.VMEM((NP,), F32) for _ in range(4)]
            + [
                pltpu.VMEM((NP,), F32),
                pltpu.VMEM((NP,), F32),
                pltpu.VMEM((NP,), F32),
                pltpu.VMEM((C,), jnp.int32),
                pltpu.VMEM((C,), jnp.int32),
                pltpu.VMEM((C,), jnp.int32),
                pltpu.VMEM((C,), jnp.int32),
                pltpu.VMEM((128,), F32),
                pltpu.VMEM((128,), F32),
                pltpu.SemaphoreType.DMA,
                pltpu.SemaphoreType.DMA,
                pltpu.SemaphoreType.DMA,
                pltpu.SemaphoreType.DMA,
            ]
        ),
    )
    def edge_kernel(tabT, alphaT, gmaxrow, src, dst, accT_o, den_o,
                    t0, t1, t2, t3, a0, a1, a2, a3,
                    asr, ads, accd, sv0, dv0, sv1, dv1, gm1, gm2,
                    ss0, sd0, ss1, sd1):
        tabs = (t0, t1, t2, t3)
        accs = (a0, a1, a2, a3)
        cid = lax.axis_index("c")
        sid = lax.axis_index("s")
        wid = sid * 2 + cid
        zeros = jnp.zeros((16,), F32)
        for t in range(units_per_tec):
            u = wid * units_per_tec + t
            head = (u // 16) if heads == 4 else (u * 0)
            setup = []
            for f in range(4):
                setup.append(pltpu.async_copy(
                    tabT.at[pl.ds((u * 4 + f) * NP, NP)], tabs[f], ss0))
            setup.append(pltpu.async_copy(
                alphaT.at[pl.ds(head * NP, NP)], asr, sd0))
            setup.append(pltpu.async_copy(
                alphaT.at[pl.ds((dst_row + head) * NP, NP)], ads, ss1))
            setup.append(pltpu.async_copy(
                gmaxrow.at[pl.ds(head * 128, 128)], gm1, sd1))
            setup.append(pltpu.async_copy(
                gmaxrow.at[pl.ds((dst_row + head) * 128, 128)], gm2, sd1))

            @plsc.parallel_loop(0, NP, 16, unroll=8)
            def _zero(o):
                for f in range(4):
                    accs[f][pl.ds(o, 16)] = zeros
                accd[pl.ds(o, 16)] = zeros

            for d in setup:
                d.wait()
            b = gm1[pl.ds(0, 16)] + gm2[pl.ds(0, 16)]
            g = jnp.maximum(b, 0.2 * b)

            def _start(ci, svb, dvb, sems):
                pltpu.async_copy(src.at[pl.ds(ci * C, C)], svb, sems[0])
                pltpu.async_copy(dst.at[pl.ds(ci * C, C)], dvb, sems[1])

            def _wait(svb, dvb, sems):
                pltpu.make_async_copy(src.at[pl.ds(0, C)], svb, sems[0]).wait()
                pltpu.make_async_copy(dst.at[pl.ds(0, C)], dvb, sems[1]).wait()

            def _run(svb, dvb, with_den):
                @plsc.parallel_loop(0, C, 16, unroll=4)
                def _body(o):
                    s = svb[pl.ds(o, 16)]
                    d = dvb[pl.ds(o, 16)]
                    e = plsc.load_gather(asr, [s]) + plsc.load_gather(ads, [d])
                    e = jnp.maximum(e, 0.2 * e)
                    ex = jnp.exp(e - g)
                    for f in range(4):
                        tv = plsc.load_gather(tabs[f], [s])
                        plsc.addupdate_scatter(accs[f], [d], tv * ex)
                    if with_den:
                        plsc.addupdate_scatter(accd, [d], ex)

            def _edge_sweep(with_den):
                _start(0, sv0, dv0, (ss0, sd0))

                def _pair(j, carry):
                    ci = 2 * j
                    _start(ci + 1, sv1, dv1, (ss1, sd1))
                    _wait(sv0, dv0, (ss0, sd0))
                    _run(sv0, dv0, with_den)
                    _start(jnp.minimum(ci + 2, chunks - 1), sv0, dv0,
                           (ss0, sd0))
                    _wait(sv1, dv1, (ss1, sd1))
                    _run(sv1, dv1, with_den)
                    return carry

                lax.fori_loop(0, chunks // 2, _pair, 0)
                _wait(sv0, dv0, (ss0, sd0))

            if heads == 4:
                is_aug = ((u == 2) | (u == 16) | (u == 34) | (u == 48))
            else:
                is_aug = (u == 0)

            @pl.when(is_aug)
            def _():
                _edge_sweep(True)

            @pl.when(jnp.logical_not(is_aug))
            def _():
                _edge_sweep(False)

            for f in range(4):
                pltpu.sync_copy(accs[f], accT_o.at[pl.ds((u * 4 + f) * NP, NP)])

            @pl.when(is_aug)
            def _():
                pltpu.sync_copy(accd, den_o.at[pl.ds(head * NP, NP)])

    return edge_kernel


_edge_l1 = _make_edge_kernel(4, 256)
_edge_l2 = _make_edge_kernel(1, 128)


def _edges(edge_index):
    loop = jnp.arange(N, dtype=edge_index.dtype)
    src = jnp.concatenate([edge_index[0], loop])
    dst = jnp.concatenate([edge_index[1], loop])
    pad = jnp.full((EP - E - N,), NP - 1, dtype=src.dtype)
    return (jnp.concatenate([src, pad]), jnp.concatenate([dst, pad]))


def _attn_mats(as1, ad1):
    eye4 = jnp.eye(4, dtype=F32)
    A1s = jnp.einsum("hc,hk->hck", as1, eye4).reshape(256, 4)
    A1d = jnp.einsum("hc,hk->hck", ad1, eye4).reshape(256, 4)
    return jnp.concatenate([A1s, A1d], axis=1)


def kernel(x0, x1, edge_index0, edge_index1, W1_0, as1_0, ad1_0, b1_0, W2_0, as2_0, ad2_0, b2_0, W1_1, as1_1, ad1_1, b1_1, W2_1, as2_1, ad2_1, b2_1):
    src0, dst0 = _edges(edge_index0)
    src1, dst1 = _edges(edge_index1)
    xp0 = jnp.pad(x0, ((0, NP - N), (0, 0)))
    xp1 = jnp.pad(x1, ((0, NP - N), (0, 0)))
    A1_0 = _attn_mats(as1_0, ad1_0)
    A1_1 = _attn_mats(as1_1, ad1_1)
    A2_0 = jnp.stack([as2_0[0], ad2_0[0]], axis=1)
    A2_1 = jnp.stack([as2_1[0], ad2_1[0]], axis=1)
    eye = jnp.eye(128, dtype=F32)

    h1T_0, alphaT_0, gmax_0 = _tc_pre(xp0, W1_0, A1_0, 256)
    h1T_1, alphaT_1, gmax_1 = _tc_pre(xp1, W1_1, A1_1, 256)
    accT0, denT0 = _edge_l1(h1T_0.reshape(-1), alphaT_0.reshape(-1),
                            gmax_0.reshape(-1), src0, dst0)
    accT1, denT1 = _edge_l1(h1T_1.reshape(-1), alphaT_1.reshape(-1),
                            gmax_1.reshape(-1), src1, dst1)
    zT0, alphaT2_0, gmax2_0 = _tc_mid(accT0.reshape(256, NP),
                                      denT0.reshape(8, NP),
                                      b1_0[:, None], W2_0, A2_0)
    zT1, alphaT2_1, gmax2_1 = _tc_mid(accT1.reshape(256, NP),
                                      denT1.reshape(8, NP),
                                      b1_1[:, None], W2_1, A2_1)
    acc2T0, den20 = _edge_l2(zT0.reshape(-1), alphaT2_0.reshape(-1),
                             gmax2_0.reshape(-1), src0, dst0)
    acc2T1, den21 = _edge_l2(zT1.reshape(-1), alphaT2_1.reshape(-1),
                             gmax2_1.reshape(-1), src1, dst1)
    out0 = _tc_post(acc2T0.reshape(128, NP), den20.reshape(8, NP),
                    b2_0[:, None], eye)
    out1 = _tc_post(acc2T1.reshape(128, NP), den21.reshape(8, NP),
                    b2_1[:, None], eye)
    return jnp.concatenate([out0[:N], out1[:N]], axis=0)


# speedup vs baseline: 1.0782x; 1.0058x over previous
import functools

import jax
import jax.numpy as jnp
from jax import lax
from jax.experimental import pallas as pl
from jax.experimental.pallas import tpu as pltpu
from jax.experimental.pallas import tpu_sc as plsc

N = 10000
NP = 10240
E = 160000
EP = 172032
C = 4096
NB = 1024
F32 = jnp.float32

_mesh = plsc.VectorSubcoreMesh(core_axis_name="c", subcore_axis_name="s")
_CP_SC = pltpu.CompilerParams(needs_layout_passes=False)


def _tc_pre_body(x_ref, w_ref, a_ref, hT_ref, al_ref, gmax_ref):
    hT = lax.dot_general(w_ref[...], x_ref[...], (((0,), (1,)), ((), ())),
                         preferred_element_type=F32)
    hT_ref[...] = hT
    al = lax.dot_general(a_ref[...], hT, (((0,), (0,)), ((), ())),
                         preferred_element_type=F32)
    al_ref[...] = al
    rm = jnp.max(al, axis=1, keepdims=True)
    rmb = lax.broadcast_in_dim(rm, (8, 128), (0, 1))

    @pl.when(pl.program_id(0) == 0)
    def _():
        gmax_ref[...] = rmb

    @pl.when(pl.program_id(0) != 0)
    def _():
        gmax_ref[...] = jnp.maximum(gmax_ref[...], rmb)


def _tc_pre(xp, W, A, dh):
    din = xp.shape[1]
    return pl.pallas_call(
        _tc_pre_body,
        grid=(NP // NB,),
        in_specs=[
            pl.BlockSpec((NB, din), lambda i: (i, 0)),
            pl.BlockSpec((din, dh), lambda i: (0, 0)),
            pl.BlockSpec((dh, 8), lambda i: (0, 0)),
        ],
        out_specs=[
            pl.BlockSpec((dh, NB), lambda i: (0, i)),
            pl.BlockSpec((8, NB), lambda i: (0, i)),
            pl.BlockSpec((8, 128), lambda i: (0, 0)),
        ],
        out_shape=[
            jax.ShapeDtypeStruct((dh, NP), F32),
            jax.ShapeDtypeStruct((8, NP), F32),
            jax.ShapeDtypeStruct((8, 128), F32),
        ],
    )(xp, W, A)


def _tc_mid_body(acc_ref, den_ref, b_ref, w_ref, a_ref,
                 zT_ref, al_ref, gmax_ref):
    i = pl.program_id(0)
    acc = acc_ref[...]
    den = den_ref[0:4, :]
    col = lax.broadcasted_iota(jnp.int32, (1, NB), 1) + i * NB
    valid = col < N
    acc = jnp.where(lax.broadcast_in_dim(valid, (256, NB), (0, 1)), acc, 0.0)
    den = jnp.where(lax.broadcast_in_dim(valid, (4, NB), (0, 1)), den, 1.0)
    acc3 = acc.reshape(4, 64, NB)
    den3 = lax.broadcast_in_dim(den, (4, 64, NB), (0, 2))
    h = acc3 / (den3 + 1e-16) + b_ref[...].reshape(4, 64, 1)
    h = h.reshape(256, NB)
    h = jnp.where(h > 0, h, jnp.exp(h) - 1.0)
    z = lax.dot_general(w_ref[...], h, (((0,), (0,)), ((), ())),
                        preferred_element_type=F32)
    zT_ref[...] = z
    al2 = lax.dot_general(a_ref[...], z, (((0,), (0,)), ((), ())),
                          preferred_element_type=F32)
    al2p = jnp.concatenate([al2, jnp.full((6, NB), -1e30, F32)], axis=0)
    al_ref[...] = al2p
    rm = jnp.max(al2p, axis=1, keepdims=True)
    rmb = lax.broadcast_in_dim(rm, (8, 128), (0, 1))

    @pl.when(i == 0)
    def _():
        gmax_ref[...] = rmb

    @pl.when(i != 0)
    def _():
        gmax_ref[...] = jnp.maximum(gmax_ref[...], rmb)


def _tc_mid(accT, denT, b1c, W2, A2):
    return pl.pallas_call(
        _tc_mid_body,
        grid=(NP // NB,),
        in_specs=[
            pl.BlockSpec((256, NB), lambda i: (0, i)),
            pl.BlockSpec((8, NB), lambda i: (0, i)),
            pl.BlockSpec((256, 1), lambda i: (0, 0)),
            pl.BlockSpec((256, 128), lambda i: (0, 0)),
            pl.BlockSpec((128, 2), lambda i: (0, 0)),
        ],
        out_specs=[
            pl.BlockSpec((128, NB), lambda i: (0, i)),
            pl.BlockSpec((8, NB), lambda i: (0, i)),
            pl.BlockSpec((8, 128), lambda i: (0, 0)),
        ],
        out_shape=[
            jax.ShapeDtypeStruct((128, NP), F32),
            jax.ShapeDtypeStruct((8, NP), F32),
            jax.ShapeDtypeStruct((8, 128), F32),
        ],
    )(accT, denT, b1c, W2, A2)


def _tc_post_body(acc_ref, den_ref, b_ref, eye_ref, out_ref):
    acc = acc_ref[...]
    den = den_ref[0:1, :]
    h = acc / (lax.broadcast_in_dim(den, (128, NB), (0, 1)) + 1e-16)
    h = h + b_ref[...]
    h = jnp.where(h > 0, h, jnp.exp(h) - 1.0)
    out_ref[...] = lax.dot_general(h, eye_ref[...], (((0,), (0,)), ((), ())),
                                   preferred_element_type=F32)


def _tc_post(acc2T, den2, b2c, eye):
    return pl.pallas_call(
        _tc_post_body,
        grid=(NP // NB,),
        in_specs=[
            pl.BlockSpec((128, NB), lambda i: (0, i)),
            pl.BlockSpec((8, NB), lambda i: (0, i)),
            pl.BlockSpec((128, 1), lambda i: (0, 0)),
            pl.BlockSpec((128, 128), lambda i: (0, 0)),
        ],
        out_specs=pl.BlockSpec((NB, 128), lambda i: (i, 0)),
        out_shape=jax.ShapeDtypeStruct((NP, 128), F32),
    )(acc2T, den2, b2c, eye)


def _make_edge_kernel(heads, featc):
    nunits = featc // 4
    units_per_tec = nunits // 32
    dst_row = 4 if heads == 4 else 1
    chunks = EP // C

    @functools.partial(
        pl.kernel,
        out_type=(jax.ShapeDtypeStruct((featc * NP,), F32),
                  jax.ShapeDtypeStruct((8 * NP,), F32)),
        mesh=_mesh,
        compiler_params=_CP_SC,
        scratch_types=(
            [pltpu.VMEM((NP,), F32) for _ in range(4)]
            + [pltpu.VMEM((NP,), F32) for _ in range(4)]
            + [
                pltpu.VMEM((NP,), F32),
                pltpu.VMEM((NP,), F32),
                pltpu.VMEM((NP,), F32),
                pltpu.VMEM((C,), jnp.int32),
                pltpu.VMEM((C,), jnp.int32),
                pltpu.VMEM((C,), jnp.int32),
                pltpu.VMEM((C,), jnp.int32),
                pltpu.VMEM((128,), F32),
                pltpu.VMEM((128,), F32),
                pltpu.SemaphoreType.DMA,
                pltpu.SemaphoreType.DMA,
                pltpu.SemaphoreType.DMA,
                pltpu.SemaphoreType.DMA,
                pltpu.SemaphoreType.DMA,
                pltpu.SemaphoreType.DMA,
            ]
        ),
    )
    def edge_kernel(tabT, alphaT, gmaxrow, src, dst, accT_o, den_o,
                    t0, t1, t2, t3, a0, a1, a2, a3,
                    asr, ads, accd, sv0, dv0, sv1, dv1, gm1, gm2,
                    ss0, sd0, ss1, sd1, su0, su1):
        tabs = (t0, t1, t2, t3)
        accs = (a0, a1, a2, a3)
        cid = lax.axis_index("c")
        sid = lax.axis_index("s")
        wid = sid * 2 + cid
        zeros = jnp.zeros((16,), F32)
        for t in range(units_per_tec):
            u = wid * units_per_tec + t
            head = (u // 16) if heads == 4 else (u * 0)
            setup = []
            for f in range(4):
                setup.append(pltpu.async_copy(
                    tabT.at[pl.ds((u * 4 + f) * NP, NP)], tabs[f], su0))
            setup.append(pltpu.async_copy(
                alphaT.at[pl.ds(head * NP, NP)], asr, su1))
            setup.append(pltpu.async_copy(
                alphaT.at[pl.ds((dst_row + head) * NP, NP)], ads, su1))
            setup.append(pltpu.async_copy(
                gmaxrow.at[pl.ds(head * 128, 128)], gm1, su1))
            setup.append(pltpu.async_copy(
                gmaxrow.at[pl.ds((dst_row + head) * 128, 128)], gm2, su1))
            pltpu.async_copy(src.at[pl.ds(0, C)], sv0, ss0)
            pltpu.async_copy(dst.at[pl.ds(0, C)], dv0, sd0)

            @plsc.parallel_loop(0, NP, 16, unroll=8)
            def _zero(o):
                for f in range(4):
                    accs[f][pl.ds(o, 16)] = zeros
                accd[pl.ds(o, 16)] = zeros

            for d in setup:
                d.wait()
            b = gm1[pl.ds(0, 16)] + gm2[pl.ds(0, 16)]
            g = jnp.maximum(b, 0.2 * b)

            def _start(ci, svb, dvb, sems):
                pltpu.async_copy(src.at[pl.ds(ci * C, C)], svb, sems[0])
                pltpu.async_copy(dst.at[pl.ds(ci * C, C)], dvb, sems[1])

            def _wait(svb, dvb, sems):
                pltpu.make_async_copy(src.at[pl.ds(0, C)], svb, sems[0]).wait()
                pltpu.make_async_copy(dst.at[pl.ds(0, C)], dvb, sems[1]).wait()

            def _run(svb, dvb, with_den):
                @plsc.parallel_loop(0, C, 16, unroll=4)
                def _body(o):
                    s = svb[pl.ds(o, 16)]
                    d = dvb[pl.ds(o, 16)]
                    e = plsc.load_gather(asr, [s]) + plsc.load_gather(ads, [d])
                    e = jnp.maximum(e, 0.2 * e)
                    ex = jnp.exp(e - g)
                    for f in range(4):
                        tv = plsc.load_gather(tabs[f], [s])
                        plsc.addupdate_scatter(accs[f], [d], tv * ex)
                    if with_den:
                        plsc.addupdate_scatter(accd, [d], ex)

            def _edge_sweep(with_den):
                def _pair(j, carry):
                    ci = 2 * j
                    _start(ci + 1, sv1, dv1, (ss1, sd1))
                    _wait(sv0, dv0, (ss0, sd0))
                    _run(sv0, dv0, with_den)
                    _start(jnp.minimum(ci + 2, chunks - 1), sv0, dv0,
                           (ss0, sd0))
                    _wait(sv1, dv1, (ss1, sd1))
                    _run(sv1, dv1, with_den)
                    return carry

                lax.fori_loop(0, chunks // 2, _pair, 0)
                _wait(sv0, dv0, (ss0, sd0))

            if heads == 4:
                is_aug = ((u == 2) | (u == 16) | (u == 34) | (u == 48))
            else:
                is_aug = (u == 0)

            @pl.when(is_aug)
            def _():
                _edge_sweep(True)

            @pl.when(jnp.logical_not(is_aug))
            def _():
                _edge_sweep(False)

            for f in range(4):
                pltpu.sync_copy(accs[f], accT_o.at[pl.ds((u * 4 + f) * NP, NP)])

            @pl.when(is_aug)
            def _():
                pltpu.sync_copy(accd, den_o.at[pl.ds(head * NP, NP)])

    return edge_kernel


_edge_l1 = _make_edge_kernel(4, 256)
_edge_l2 = _make_edge_kernel(1, 128)


def _edges(edge_index):
    loop = jnp.arange(N, dtype=edge_index.dtype)
    src = jnp.concatenate([edge_index[0], loop])
    dst = jnp.concatenate([edge_index[1], loop])
    pad = jnp.full((EP - E - N,), NP - 1, dtype=src.dtype)
    return (jnp.concatenate([src, pad]), jnp.concatenate([dst, pad]))


def _attn_mats(as1, ad1):
    eye4 = jnp.eye(4, dtype=F32)
    A1s = jnp.einsum("hc,hk->hck", as1, eye4).reshape(256, 4)
    A1d = jnp.einsum("hc,hk->hck", ad1, eye4).reshape(256, 4)
    return jnp.concatenate([A1s, A1d], axis=1)


def kernel(x0, x1, edge_index0, edge_index1, W1_0, as1_0, ad1_0, b1_0, W2_0, as2_0, ad2_0, b2_0, W1_1, as1_1, ad1_1, b1_1, W2_1, as2_1, ad2_1, b2_1):
    src0, dst0 = _edges(edge_index0)
    src1, dst1 = _edges(edge_index1)
    xp0 = jnp.pad(x0, ((0, NP - N), (0, 0)))
    xp1 = jnp.pad(x1, ((0, NP - N), (0, 0)))
    A1_0 = _attn_mats(as1_0, ad1_0)
    A1_1 = _attn_mats(as1_1, ad1_1)
    A2_0 = jnp.stack([as2_0[0], ad2_0[0]], axis=1)
    A2_1 = jnp.stack([as2_1[0], ad2_1[0]], axis=1)
    eye = jnp.eye(128, dtype=F32)

    h1T_0, alphaT_0, gmax_0 = _tc_pre(xp0, W1_0, A1_0, 256)
    h1T_1, alphaT_1, gmax_1 = _tc_pre(xp1, W1_1, A1_1, 256)
    accT0, denT0 = _edge_l1(h1T_0.reshape(-1), alphaT_0.reshape(-1),
                            gmax_0.reshape(-1), src0, dst0)
    accT1, denT1 = _edge_l1(h1T_1.reshape(-1), alphaT_1.reshape(-1),
                            gmax_1.reshape(-1), src1, dst1)
    zT0, alphaT2_0, gmax2_0 = _tc_mid(accT0.reshape(256, NP),
                                      denT0.reshape(8, NP),
                                      b1_0[:, None], W2_0, A2_0)
    zT1, alphaT2_1, gmax2_1 = _tc_mid(accT1.reshape(256, NP),
                                      denT1.reshape(8, NP),
                                      b1_1[:, None], W2_1, A2_1)
    acc2T0, den20 = _edge_l2(zT0.reshape(-1), alphaT2_0.reshape(-1),
                             gmax2_0.reshape(-1), src0, dst0)
    acc2T1, den21 = _edge_l2(zT1.reshape(-1), alphaT2_1.reshape(-1),
                             gmax2_1.reshape(-1), src1, dst1)
    out0 = _tc_post(acc2T0.reshape(128, NP), den20.reshape(8, NP),
                    b2_0[:, None], eye)
    out1 = _tc_post(acc2T1.reshape(128, NP), den21.reshape(8, NP),
                    b2_1[:, None], eye)
    return jnp.concatenate([out0[:N], out1[:N]], axis=0)
